# baseline (speedup 1.0000x reference)
import functools

import jax
import jax.numpy as jnp
from jax import lax
from jax.experimental import pallas as pl
from jax.experimental.pallas import tpu as pltpu
from jax.experimental.pallas import tpu_sc as plsc

NUM_TRICKS = 100000
NUM_DRAFT_TRICKS = 1000
NUM_EMBEDDINGS = NUM_TRICKS + NUM_DRAFT_TRICKS
DRAFT_DELTA = NUM_TRICKS
DRAFT_PHASE = 2
EMBED_DIM = 128

NUM_WORKERS = 32
LANES = 16
CHUNK = 128
NBUF = 5


@functools.lru_cache(maxsize=None)
def _build(n_total):
    n = n_total // NUM_WORKERS
    n_units = n // CHUNK
    rounds = n_units // NBUF
    vec_per_unit = CHUNK // LANES
    mesh = plsc.VectorSubcoreMesh(core_axis_name="c", subcore_axis_name="s")

    @functools.partial(
        pl.kernel,
        mesh=mesh,
        out_type=jax.ShapeDtypeStruct((n_total, EMBED_DIM), jnp.float32),
        scratch_types=[
            pltpu.VMEM((n,), jnp.int32),
            pltpu.VMEM((n,), jnp.int32),
        ]
        + [pltpu.VMEM((CHUNK, EMBED_DIM), jnp.float32) for _ in range(NBUF)]
        + [pltpu.SemaphoreType.DMA for _ in range(2 * NBUF)],
    )
    def kern(trick_hbm, phase_hbm, table_hbm, out_hbm, idx_v, phase_v, *bufs_sems):
        rows = bufs_sems[:NBUF]
        g_sem = bufs_sems[NBUF:2 * NBUF]
        s_sem = bufs_sems[2 * NBUF:]
        wid = lax.axis_index("s") * 2 + lax.axis_index("c")
        base = wid * n
        pltpu.sync_copy(trick_hbm.at[pl.ds(base, n)], idx_v)
        pltpu.sync_copy(phase_hbm.at[pl.ds(base, n)], phase_v)

        def compute_idx(u):
            for j in range(vec_per_unit):
                o = u * CHUNK + j * LANES
                t = idx_v[pl.ds(o, LANES)]
                p = phase_v[pl.ds(o, LANES)]
                t = t + jnp.where(p == DRAFT_PHASE, DRAFT_DELTA, 0)
                idx_v[pl.ds(o, LANES)] = jnp.clip(t, -1, NUM_EMBEDDINGS - 1) + 1

        def gather_start(u, b):
            pltpu.async_copy(
                table_hbm.at[idx_v.at[pl.ds(u * CHUNK, CHUNK)]], rows[b], g_sem[b]
            )

        def gather_wait(u, b):
            pltpu.make_async_copy(
                table_hbm.at[idx_v.at[pl.ds(u * CHUNK, CHUNK)]], rows[b], g_sem[b]
            ).wait()

        def scatter_start(u, b):
            return pltpu.async_copy(
                rows[b], out_hbm.at[pl.ds(base + u * CHUNK, CHUNK)], s_sem[b]
            )

        for b in range(NBUF):
            compute_idx(b)
            gather_start(b, b)

        def main_round(r, carry):
            u0 = r * NBUF
            for b in range(NBUF):
                u = u0 + b
                gather_wait(u, b)
                sc = scatter_start(u, b)
                compute_idx(u + NBUF)
                sc.wait()
                gather_start(u + NBUF, b)
            return carry

        lax.fori_loop(0, rounds - 1, main_round, 0)

        u0 = (rounds - 1) * NBUF
        for b in range(NBUF):
            u = u0 + b
            gather_wait(u, b)
            scatter_start(u, b).wait()

    return kern


def kernel(trick, phase, table):
    b, h = trick.shape
    n_total = b * h
    out = _build(n_total)(
        trick.reshape(n_total).astype(jnp.int32),
        phase.reshape(n_total).astype(jnp.int32),
        table,
    )
    return out.reshape(b, h, EMBED_DIM)


# speedup vs baseline: 1.0015x; 1.0015x over previous
import functools

import jax
import jax.numpy as jnp
from jax import lax
from jax.experimental import pallas as pl
from jax.experimental.pallas import tpu as pltpu
from jax.experimental.pallas import tpu_sc as plsc

NUM_TRICKS = 100000
NUM_DRAFT_TRICKS = 1000
NUM_EMBEDDINGS = NUM_TRICKS + NUM_DRAFT_TRICKS
DRAFT_DELTA = NUM_TRICKS
DRAFT_PHASE = 2
EMBED_DIM = 128

NUM_WORKERS = 32
LANES = 16
CHUNK = 128
NBUF = 5


@functools.lru_cache(maxsize=None)
def _build(n_total):
    n = n_total // NUM_WORKERS
    n_units = n // CHUNK
    rounds = n_units // NBUF
    vec_per_unit = CHUNK // LANES
    mesh = plsc.VectorSubcoreMesh(core_axis_name="c", subcore_axis_name="s")

    @functools.partial(
        pl.kernel,
        mesh=mesh,
        compiler_params=pltpu.CompilerParams(use_tc_tiling_on_sc=False),
        out_type=jax.ShapeDtypeStruct((n_total, EMBED_DIM), jnp.float32),
        scratch_types=[
            pltpu.VMEM((n,), jnp.int32),
            pltpu.VMEM((n,), jnp.int32),
        ]
        + [pltpu.VMEM((CHUNK, EMBED_DIM), jnp.float32) for _ in range(NBUF)]
        + [pltpu.SemaphoreType.DMA for _ in range(2 * NBUF)],
    )
    def kern(trick_hbm, phase_hbm, table_hbm, out_hbm, idx_v, phase_v, *bufs_sems):
        rows = bufs_sems[:NBUF]
        g_sem = bufs_sems[NBUF:2 * NBUF]
        s_sem = bufs_sems[2 * NBUF:]
        wid = lax.axis_index("s") * 2 + lax.axis_index("c")
        base = wid * n
        pltpu.sync_copy(trick_hbm.at[pl.ds(base, n)], idx_v)
        pltpu.sync_copy(phase_hbm.at[pl.ds(base, n)], phase_v)

        def compute_idx(u):
            for j in range(vec_per_unit):
                o = u * CHUNK + j * LANES
                t = idx_v[pl.ds(o, LANES)]
                p = phase_v[pl.ds(o, LANES)]
                t = t + jnp.where(p == DRAFT_PHASE, DRAFT_DELTA, 0)
                idx_v[pl.ds(o, LANES)] = jnp.clip(t, -1, NUM_EMBEDDINGS - 1) + 1

        def gather_start(u, b):
            pltpu.async_copy(
                table_hbm.at[idx_v.at[pl.ds(u * CHUNK, CHUNK)]], rows[b], g_sem[b]
            )

        def gather_wait(u, b):
            pltpu.make_async_copy(
                table_hbm.at[idx_v.at[pl.ds(u * CHUNK, CHUNK)]], rows[b], g_sem[b]
            ).wait()

        def scatter_start(u, b):
            return pltpu.async_copy(
                rows[b], out_hbm.at[pl.ds(base + u * CHUNK, CHUNK)], s_sem[b]
            )

        for b in range(NBUF):
            compute_idx(b)
            gather_start(b, b)

        def main_round(r, carry):
            u0 = r * NBUF
            for b in range(NBUF):
                u = u0 + b
                gather_wait(u, b)
                sc = scatter_start(u, b)
                compute_idx(u + NBUF)
                sc.wait()
                gather_start(u + NBUF, b)
            return carry

        lax.fori_loop(0, rounds - 1, main_round, 0)

        u0 = (rounds - 1) * NBUF
        for b in range(NBUF):
            u = u0 + b
            gather_wait(u, b)
            scatter_start(u, b).wait()

    return kern


def kernel(trick, phase, table):
    b, h = trick.shape
    n_total = b * h
    out = _build(n_total)(
        trick.reshape(n_total).astype(jnp.int32),
        phase.reshape(n_total).astype(jnp.int32),
        table,
    )
    return out.reshape(b, h, EMBED_DIM)


# speedup vs baseline: 3.3936x; 3.3886x over previous
import functools

import jax
import jax.numpy as jnp
from jax import lax
from jax.experimental import pallas as pl
from jax.experimental.pallas import tpu as pltpu
from jax.experimental.pallas import tpu_sc as plsc

NUM_TRICKS = 100000
NUM_DRAFT_TRICKS = 1000
NUM_EMBEDDINGS = NUM_TRICKS + NUM_DRAFT_TRICKS
DRAFT_DELTA = NUM_TRICKS
DRAFT_PHASE = 2
EMBED_DIM = 128

V = NUM_EMBEDDINGS + 1
HALF = EMBED_DIM // 2
LANES = 16
NSUB = 16
S = 1600
SUPERS = 8
R = 5120
W = 20
SENT = -1
GROUPS = (12 * (128,) + (64,))


@functools.lru_cache(maxsize=None)
def _build(n_total):
    n_tile = n_total // NSUB
    supers = n_tile // S
    mesh = plsc.VectorSubcoreMesh(core_axis_name="c", subcore_axis_name="s")

    @functools.partial(
        pl.kernel,
        mesh=mesh,
        compiler_params=pltpu.CompilerParams(use_tc_tiling_on_sc=False),
        out_type=jax.ShapeDtypeStruct((n_total, EMBED_DIM), jnp.float32),
        scratch_types=[
            pltpu.VMEM((S,), jnp.int32),
            pltpu.VMEM((S,), jnp.int32),
            pltpu.VMEM((S,), jnp.int32),
            pltpu.VMEM((S, HALF), jnp.float32),
            pltpu.VMEM((LANES,), jnp.int32),
            pltpu.VMEM((8, HALF), jnp.float32),
            pltpu.VMEM_SHARED((R, HALF), jnp.float32),
            pltpu.SemaphoreType.DMA,
            pltpu.SemaphoreType.DMA,
            pltpu.SemaphoreType.DMA,
        ],
    )
    def kern(trick_hbm, phase_hbm, table_hbm, out_hbm,
             trick_v, phase_v, wlist_v, chunk_v, fidx_v, scrap_v,
             win_sh, lsem, gsem, fsem):
        c = lax.axis_index("c")
        s = lax.axis_index("s")
        col0 = c * HALF

        def super_body(sp, carry):
            fidx_v[pl.ds(0, LANES)] = jnp.zeros((LANES,), jnp.int32)
            pbase = s * n_tile + sp * S
            pltpu.sync_copy(trick_hbm.at[pl.ds(pbase, S)], trick_v)
            pltpu.sync_copy(phase_hbm.at[pl.ds(pbase, S)], phase_v)

            def adj(j, cr):
                o = j * LANES
                t = trick_v[pl.ds(o, LANES)]
                p = phase_v[pl.ds(o, LANES)]
                t = t + jnp.where(p == DRAFT_PHASE, DRAFT_DELTA, 0)
                trick_v[pl.ds(o, LANES)] = (
                    jnp.clip(t, -1, NUM_EMBEDDINGS - 1) + 1
                )
                return cr

            lax.fori_loop(0, S // LANES, adj, 0)

            for w in range(W):
                wbase = w * R
                plsc.subcore_barrier()
                rows_full = R // NSUB

                def load(nrows):
                    pltpu.async_copy(
                        table_hbm.at[
                            pl.ds(wbase + s * rows_full, nrows),
                            pl.ds(col0, HALF),
                        ],
                        win_sh.at[pl.ds(s * rows_full, nrows)],
                        lsem,
                    ).wait()

                if w < W - 1:
                    load(rows_full)
                else:
                    last_rows = V - (W - 1) * R
                    full_tiles = last_rows // rows_full
                    rem = last_rows % rows_full

                    @pl.when(s < full_tiles)
                    def _():
                        load(rows_full)

                    if rem:
                        @pl.when(s == full_tiles)
                        def _():
                            load(rem)

                plsc.subcore_barrier()

                def bld(j, acc):
                    o = j * LANES
                    v = trick_v[pl.ds(o, LANES)] - wbase
                    m = jnp.logical_and(v >= 0, v < R)
                    wlist_v[pl.ds(o, LANES)] = jnp.where(m, v, SENT)
                    return acc

                lax.fori_loop(0, S // LANES, bld, 0)

                goff = 0
                for gsz in GROUPS:
                    pltpu.async_copy(
                        win_sh.at[
                            plsc.Indices(
                                wlist_v.at[pl.ds(goff, gsz)],
                                ignored_value=SENT,
                            )
                        ],
                        chunk_v.at[pl.ds(goff, gsz)],
                        gsem,
                    )
                    goff += gsz
                pass

            pltpu.make_async_copy(
                table_hbm.at[pl.ds(0, S), pl.ds(col0, HALF)], chunk_v, gsem
            ).wait()
            pltpu.sync_copy(
                chunk_v, out_hbm.at[pl.ds(pbase, S), pl.ds(col0, HALF)]
            )
            return carry

        lax.fori_loop(0, supers, super_body, 0)

    return kern


def kernel(trick, phase, table):
    b, h = trick.shape
    n_total = b * h
    out = _build(n_total)(
        trick.reshape(n_total).astype(jnp.int32),
        phase.reshape(n_total).astype(jnp.int32),
        table,
    )
    return out.reshape(b, h, EMBED_DIM)


# speedup vs baseline: 3.8094x; 1.1225x over previous
import functools

import jax
import jax.numpy as jnp
from jax import lax
from jax.experimental import pallas as pl
from jax.experimental.pallas import tpu as pltpu
from jax.experimental.pallas import tpu_sc as plsc

NUM_TRICKS = 100000
NUM_DRAFT_TRICKS = 1000
NUM_EMBEDDINGS = NUM_TRICKS + NUM_DRAFT_TRICKS
DRAFT_DELTA = NUM_TRICKS
DRAFT_PHASE = 2
EMBED_DIM = 128

V = NUM_EMBEDDINGS + 1
HALF = EMBED_DIM // 2
LANES = 16
NSUB = 16
S = 1600
SUPERS = 8
R = 5824
W = 18
SENT = -1
GROUPS = (12 * (128,) + (64,))


@functools.lru_cache(maxsize=None)
def _build(n_total):
    n_tile = n_total // NSUB
    supers = n_tile // S
    mesh = plsc.VectorSubcoreMesh(core_axis_name="c", subcore_axis_name="s")

    @functools.partial(
        pl.kernel,
        mesh=mesh,
        compiler_params=pltpu.CompilerParams(use_tc_tiling_on_sc=False),
        out_type=jax.ShapeDtypeStruct((n_total, EMBED_DIM), jnp.float32),
        scratch_types=[
            pltpu.VMEM((S,), jnp.int32),
            pltpu.VMEM((S,), jnp.int32),
            pltpu.VMEM((S,), jnp.int32),
            pltpu.VMEM((S, HALF), jnp.float32),
            pltpu.VMEM((LANES,), jnp.int32),
            pltpu.VMEM((8, HALF), jnp.float32),
            pltpu.VMEM_SHARED((R, HALF), jnp.float32),
            pltpu.SemaphoreType.DMA,
            pltpu.SemaphoreType.DMA,
            pltpu.SemaphoreType.DMA,
            pltpu.SemaphoreType.DMA,
        ],
    )
    def kern(trick_hbm, phase_hbm, table_hbm, out_hbm,
             trick_v, phase_v, wlist_v, chunk_v, fidx_v, scrap_v,
             win_sh, lsem, gsem, fsem, ssem):
        c = lax.axis_index("c")
        s = lax.axis_index("s")
        col0 = c * HALF

        def super_body(sp, carry):
            fidx_v[pl.ds(0, LANES)] = jnp.zeros((LANES,), jnp.int32)
            pbase = s * n_tile + sp * S
            pltpu.sync_copy(trick_hbm.at[pl.ds(pbase, S)], trick_v)
            pltpu.sync_copy(phase_hbm.at[pl.ds(pbase, S)], phase_v)

            def adj(j, cr):
                o = j * LANES
                t = trick_v[pl.ds(o, LANES)]
                p = phase_v[pl.ds(o, LANES)]
                t = t + jnp.where(p == DRAFT_PHASE, DRAFT_DELTA, 0)
                trick_v[pl.ds(o, LANES)] = (
                    jnp.clip(t, -1, NUM_EMBEDDINGS - 1) + 1
                )
                return cr

            lax.fori_loop(0, S // LANES, adj, 0)

            for w in range(W):
                wbase = w * R
                plsc.subcore_barrier()
                rows_full = R // NSUB

                def load(nrows):
                    pltpu.async_copy(
                        table_hbm.at[
                            pl.ds(wbase + s * rows_full, nrows),
                            pl.ds(col0, HALF),
                        ],
                        win_sh.at[pl.ds(s * rows_full, nrows)],
                        lsem,
                    ).wait()

                if w < W - 1:
                    load(rows_full)
                else:
                    last_rows = V - (W - 1) * R
                    full_tiles = last_rows // rows_full
                    rem = last_rows % rows_full

                    @pl.when(s < full_tiles)
                    def _():
                        load(rows_full)

                    if rem:
                        @pl.when(s == full_tiles)
                        def _():
                            load(rem)

                plsc.subcore_barrier()

                def bld(j, acc):
                    o = j * LANES
                    v = trick_v[pl.ds(o, LANES)] - wbase
                    m = jnp.logical_and(v >= 0, v < R)
                    wlist_v[pl.ds(o, LANES)] = jnp.where(m, v, SENT)
                    return acc

                lax.fori_loop(0, S // LANES, bld, 0)

                if w == 0:
                    @pl.when(sp > 0)
                    def _():
                        pltpu.make_async_copy(
                            chunk_v,
                            out_hbm.at[pl.ds(pbase, S), pl.ds(col0, HALF)],
                            ssem,
                        ).wait()

                goff = 0
                for gsz in GROUPS:
                    pltpu.async_copy(
                        win_sh.at[
                            plsc.Indices(
                                wlist_v.at[pl.ds(goff, gsz)],
                                ignored_value=SENT,
                            )
                        ],
                        chunk_v.at[pl.ds(goff, gsz)],
                        gsem,
                    )
                    goff += gsz
                pltpu.async_copy(
                    win_sh.at[plsc.Indices(fidx_v.at[pl.ds(0, 8)])],
                    scrap_v,
                    fsem,
                ).wait()

            pltpu.make_async_copy(
                table_hbm.at[pl.ds(0, S), pl.ds(col0, HALF)], chunk_v, gsem
            ).wait()
            pltpu.async_copy(
                chunk_v, out_hbm.at[pl.ds(pbase, S), pl.ds(col0, HALF)], ssem
            )
            return carry

        lax.fori_loop(0, supers, super_body, 0)
        pltpu.make_async_copy(
            chunk_v,
            out_hbm.at[pl.ds(s * n_tile + (supers - 1) * S, S),
                       pl.ds(col0, HALF)],
            ssem,
        ).wait()

    return kern


def kernel(trick, phase, table):
    b, h = trick.shape
    n_total = b * h
    out = _build(n_total)(
        trick.reshape(n_total).astype(jnp.int32),
        phase.reshape(n_total).astype(jnp.int32),
        table,
    )
    return out.reshape(b, h, EMBED_DIM)
